# Initial kernel scaffold; baseline (speedup 1.0000x reference)
import functools

import jax
import jax.numpy as jnp
from jax import lax
from jax.experimental import pallas as pl
from jax.experimental.pallas import tpu as pltpu
from jax.experimental.pallas import tpu_sc as plsc

N = 10000
E = 320000
NP = 10240
NC = 2
NS = 16
NW = NC * NS
RZ = NP // NS
C = 128
CH = 80
EP = NW * CH * C
DW = 16
BR = 512

_mesh = plsc.VectorSubcoreMesh(core_axis_name="c", subcore_axis_name="s")


def _make_deg():
    @functools.partial(
        pl.kernel,
        out_type=jax.ShapeDtypeStruct((NC * NP, DW), jnp.float32),
        mesh=_mesh,
        scratch_types=[
            pltpu.VMEM((C,), jnp.int32),
            pltpu.VMEM((C, DW), jnp.float32),
            pltpu.VMEM_SHARED((NP, DW), jnp.float32),
            pltpu.SemaphoreType.DMA,
        ],
    )
    def deg_kernel(dst_hbm, ones_hbm, zeros_hbm, out_hbm, dstv, onesv, acc, sem):
        c = lax.axis_index("c")
        s = lax.axis_index("s")
        w = c * NS + s
        pltpu.sync_copy(ones_hbm, onesv)
        pltpu.sync_copy(zeros_hbm, acc.at[pl.ds(s * RZ, RZ)])
        plsc.subcore_barrier()

        @pl.loop(0, CH)
        def _(j):
            base = (w * CH + j) * C
            pltpu.sync_copy(dst_hbm.at[pl.ds(base, C)], dstv)
            pltpu.sync_copy(onesv, acc.at[dstv], add=True)

        plsc.subcore_barrier()
        pltpu.sync_copy(acc.at[pl.ds(s * RZ, RZ)],
                        out_hbm.at[pl.ds(c * NP + s * RZ, RZ)])

    return deg_kernel


def _make_agg(d):
    @functools.partial(
        pl.kernel,
        out_type=jax.ShapeDtypeStruct((NC * NP, d), jnp.float32),
        mesh=_mesh,
        scratch_types=[
            pltpu.VMEM((C,), jnp.int32),
            pltpu.VMEM((C,), jnp.int32),
            pltpu.VMEM((C, d), jnp.float32),
            pltpu.VMEM_SHARED((NP, d), jnp.float32),
            pltpu.SemaphoreType.DMA,
        ],
    )
    def agg_kernel(g_hbm, src_hbm, dst_hbm, zeros_hbm, out_hbm,
                   srcv, dstv, buf, acc, sem):
        c = lax.axis_index("c")
        s = lax.axis_index("s")
        w = c * NS + s
        pltpu.sync_copy(zeros_hbm, acc.at[pl.ds(s * RZ, RZ)])
        plsc.subcore_barrier()

        @pl.loop(0, CH)
        def _(j):
            base = (w * CH + j) * C
            pltpu.sync_copy(src_hbm.at[pl.ds(base, C)], srcv)
            pltpu.sync_copy(dst_hbm.at[pl.ds(base, C)], dstv)
            pltpu.async_copy(g_hbm.at[srcv], buf, sem).wait()
            pltpu.sync_copy(buf, acc.at[dstv], add=True)

        plsc.subcore_barrier()
        pltpu.sync_copy(acc.at[pl.ds(s * RZ, RZ)],
                        out_hbm.at[pl.ds(c * NP + s * RZ, RZ)])

    return agg_kernel


_deg_call = _make_deg()
_agg_call = {128: _make_agg(128), 64: _make_agg(64)}


def _matmul(x, w):
    k, m = w.shape

    def body(x_ref, w_ref, o_ref):
        o_ref[...] = jnp.dot(x_ref[...], w_ref[...],
                             preferred_element_type=jnp.float32)

    return pl.pallas_call(
        body,
        grid=(NP // BR,),
        in_specs=[
            pl.BlockSpec((BR, k), lambda i: (i, 0)),
            pl.BlockSpec((k, m), lambda i: (0, 0)),
        ],
        out_specs=pl.BlockSpec((BR, m), lambda i: (i, 0)),
        out_shape=jax.ShapeDtypeStruct((NP, m), jnp.float32),
    )(x, w)


def _dinv_scale(degp, h0):
    def body(degp_ref, h_ref, dinv_ref, g_ref):
        deg = degp_ref[0, :, 0] + degp_ref[1, :, 0] + 1.0
        dinv = lax.rsqrt(deg)
        dinv_ref[...] = jnp.broadcast_to(dinv[:, None], (BR, 128))
        g_ref[...] = dinv[:, None] * h_ref[...]

    return pl.pallas_call(
        body,
        grid=(NP // BR,),
        in_specs=[
            pl.BlockSpec((2, BR, DW), lambda i: (0, i, 0)),
            pl.BlockSpec((BR, 128), lambda i: (i, 0)),
        ],
        out_specs=[
            pl.BlockSpec((BR, 128), lambda i: (i, 0)),
            pl.BlockSpec((BR, 128), lambda i: (i, 0)),
        ],
        out_shape=[jax.ShapeDtypeStruct((NP, 128), jnp.float32)] * 2,
    )(degp, h0)


def _finalize_matmul(ap, g, dinv, b, wnext):
    d = g.shape[1]
    k, m = wnext.shape

    def body(ap_ref, g_ref, dinv_ref, b_ref, w_ref, x_ref, gn_ref):
        ssum = ap_ref[0] + ap_ref[1] + g_ref[...]
        xl = jnp.maximum(dinv_ref[:, :d] * ssum + b_ref[...], 0.0)
        x_ref[...] = xl
        gn_ref[...] = dinv_ref[:, :m] * jnp.dot(
            xl, w_ref[...], preferred_element_type=jnp.float32)

    return pl.pallas_call(
        body,
        grid=(NP // BR,),
        in_specs=[
            pl.BlockSpec((2, BR, d), lambda i: (0, i, 0)),
            pl.BlockSpec((BR, d), lambda i: (i, 0)),
            pl.BlockSpec((BR, 128), lambda i: (i, 0)),
            pl.BlockSpec((1, d), lambda i: (0, 0)),
            pl.BlockSpec((k, m), lambda i: (0, 0)),
        ],
        out_specs=[
            pl.BlockSpec((BR, d), lambda i: (i, 0)),
            pl.BlockSpec((BR, m), lambda i: (i, 0)),
        ],
        out_shape=[
            jax.ShapeDtypeStruct((NP, d), jnp.float32),
            jax.ShapeDtypeStruct((NP, m), jnp.float32),
        ],
    )(ap, g, dinv, b, wnext)


def _finalize_residual(ap, g, dinv, b, x0):
    def body(ap_ref, g_ref, dinv_ref, b_ref, x0_ref, o_ref):
        ssum = ap_ref[0] + ap_ref[1] + g_ref[...]
        o_ref[...] = jnp.maximum(
            dinv_ref[...] * ssum + b_ref[...] + x0_ref[...], 0.0)

    return pl.pallas_call(
        body,
        grid=(NP // BR,),
        in_specs=[
            pl.BlockSpec((2, BR, 128), lambda i: (0, i, 0)),
            pl.BlockSpec((BR, 128), lambda i: (i, 0)),
            pl.BlockSpec((BR, 128), lambda i: (i, 0)),
            pl.BlockSpec((1, 128), lambda i: (0, 0)),
            pl.BlockSpec((BR, 128), lambda i: (i, 0)),
        ],
        out_specs=pl.BlockSpec((BR, 128), lambda i: (i, 0)),
        out_shape=jax.ShapeDtypeStruct((NP, 128), jnp.float32),
    )(ap, g, dinv, b, x0)


def kernel(x, edge_index, W0, b0, W1, b1, W2, b2):
    src = edge_index[0]
    dst = edge_index[1]
    npad = EP - E
    srcp = jnp.concatenate([src, jnp.full((npad,), N, jnp.int32)])
    dstp = jnp.concatenate([dst, jnp.full((npad,), NP - 1, jnp.int32)])
    xp = jnp.pad(x, ((0, NP - N), (0, 0)))

    ones_deg = jnp.ones((C, DW), jnp.float32)
    zeros_deg = jnp.zeros((RZ, DW), jnp.float32)
    zeros128 = jnp.zeros((RZ, 128), jnp.float32)
    zeros64 = jnp.zeros((RZ, 64), jnp.float32)

    degp = _deg_call(dstp, ones_deg, zeros_deg).reshape(NC, NP, DW)
    h0 = _matmul(xp, W0)
    dinv, g0 = _dinv_scale(degp, h0)

    a0 = _agg_call[128](g0, srcp, dstp, zeros128).reshape(NC, NP, 128)
    x0, g1 = _finalize_matmul(a0, g0, dinv, b0.reshape(1, 128), W1)

    a1 = _agg_call[64](g1, srcp, dstp, zeros64).reshape(NC, NP, 64)
    _, g2 = _finalize_matmul(a1, g1, dinv, b1.reshape(1, 64), W2)

    a2 = _agg_call[128](g2, srcp, dstp, zeros128).reshape(NC, NP, 128)
    out = _finalize_residual(a2, g2, dinv, b2.reshape(1, 128), x0)
    return out[:N]


# speedup vs baseline: 6.0478x; 6.0478x over previous
import dataclasses
import functools

import jax
import jax.numpy as jnp
from jax import lax
from jax.experimental import pallas as pl
from jax.experimental.pallas import tpu as pltpu
from jax.experimental.pallas import tpu_sc as plsc

N = 10000
E = 320000
NP = 10240
NC = 2
NS = 16
NW = NC * NS
RZ = NP // NS
C = 128
CH = 80
EP = NW * CH * C
BR = 512

_mesh = plsc.VectorSubcoreMesh(core_axis_name="c", subcore_axis_name="s")

_sc_params = pltpu.CompilerParams()
if "needs_layout_passes" in pltpu.CompilerParams.__dataclass_fields__:
    _sc_params = dataclasses.replace(_sc_params, needs_layout_passes=False)


def _make_deg():
    @functools.partial(
        pl.kernel,
        out_type=jax.ShapeDtypeStruct((NW, NP), jnp.float32),
        mesh=_mesh,
        compiler_params=_sc_params,
        scratch_types=[
            pltpu.VMEM((C,), jnp.int32),
            pltpu.VMEM((NP,), jnp.float32),
            pltpu.SemaphoreType.DMA,
        ],
    )
    def deg_kernel(dst_hbm, zeros_hbm, out_hbm, dstv, degloc, sem):
        c = lax.axis_index("c")
        s = lax.axis_index("s")
        w = c * NS + s
        pltpu.sync_copy(zeros_hbm, degloc)
        ones16 = jnp.ones((16,), jnp.float32)

        @pl.loop(0, CH)
        def _(j):
            base = (w * CH + j) * C
            pltpu.sync_copy(dst_hbm.at[pl.ds(base, C)], dstv)
            for k in range(C // 16):
                idx = dstv[pl.ds(k * 16, 16)]
                plsc.addupdate_scatter(degloc, [idx], ones16)

        pltpu.sync_copy(degloc, out_hbm.at[w])

    return deg_kernel


def _make_agg(d):
    @functools.partial(
        pl.kernel,
        out_type=jax.ShapeDtypeStruct((NC * NP, d), jnp.float32),
        mesh=_mesh,
        scratch_types=[
            pltpu.VMEM((C,), jnp.int32),
            pltpu.VMEM((C,), jnp.int32),
            pltpu.VMEM((C, d), jnp.float32),
            pltpu.VMEM_SHARED((NP, d), jnp.float32),
            pltpu.SemaphoreType.DMA,
        ],
    )
    def agg_kernel(g_hbm, src_hbm, dst_hbm, zeros_hbm, out_hbm,
                   srcv, dstv, buf, acc, sem):
        c = lax.axis_index("c")
        s = lax.axis_index("s")
        w = c * NS + s
        pltpu.sync_copy(zeros_hbm, acc.at[pl.ds(s * RZ, RZ)])
        plsc.subcore_barrier()

        @pl.loop(0, CH)
        def _(j):
            base = (w * CH + j) * C
            pltpu.sync_copy(src_hbm.at[pl.ds(base, C)], srcv)
            pltpu.sync_copy(dst_hbm.at[pl.ds(base, C)], dstv)
            pltpu.async_copy(g_hbm.at[srcv], buf, sem).wait()
            pltpu.sync_copy(buf, acc.at[dstv], add=True)

        plsc.subcore_barrier()
        pltpu.sync_copy(acc.at[pl.ds(s * RZ, RZ)],
                        out_hbm.at[pl.ds(c * NP + s * RZ, RZ)])

    return agg_kernel


_deg_call = _make_deg()
_agg_call = _make_agg(128)


def _matmul(x, w):
    k, m = w.shape

    def body(x_ref, w_ref, o_ref):
        o_ref[...] = jnp.dot(x_ref[...], w_ref[...],
                             preferred_element_type=jnp.float32)

    return pl.pallas_call(
        body,
        grid=(NP // BR,),
        in_specs=[
            pl.BlockSpec((BR, k), lambda i: (i, 0)),
            pl.BlockSpec((k, m), lambda i: (0, 0)),
        ],
        out_specs=pl.BlockSpec((BR, m), lambda i: (i, 0)),
        out_shape=jax.ShapeDtypeStruct((NP, m), jnp.float32),
    )(x, w)


def _dinv_scale(degp, h0):
    def body(degp_ref, h_ref, dinv_ref, g_ref):
        deg = jnp.sum(degp_ref[...], axis=1, keepdims=True) + 1.0
        dinv = lax.rsqrt(deg)
        dinv_ref[...] = jnp.broadcast_to(dinv, (BR, 128))
        g_ref[...] = dinv * h_ref[...]

    return pl.pallas_call(
        body,
        grid=(NP // BR,),
        in_specs=[
            pl.BlockSpec((BR, NW), lambda i: (i, 0)),
            pl.BlockSpec((BR, 128), lambda i: (i, 0)),
        ],
        out_specs=[
            pl.BlockSpec((BR, 128), lambda i: (i, 0)),
            pl.BlockSpec((BR, 128), lambda i: (i, 0)),
        ],
        out_shape=[jax.ShapeDtypeStruct((NP, 128), jnp.float32)] * 2,
    )(degp, h0)


def _finalize_matmul(ap, g, dinv, b, wnext):
    d = g.shape[1]
    k, m = wnext.shape

    def body(ap_ref, g_ref, dinv_ref, b_ref, w_ref, x_ref, gn_ref):
        ssum = ap_ref[0] + ap_ref[1] + g_ref[...]
        xl = jnp.maximum(dinv_ref[:, :d] * ssum + b_ref[...], 0.0)
        x_ref[...] = xl
        gn_ref[...] = dinv_ref[:, :m] * jnp.dot(
            xl, w_ref[...], preferred_element_type=jnp.float32)

    return pl.pallas_call(
        body,
        grid=(NP // BR,),
        in_specs=[
            pl.BlockSpec((2, BR, d), lambda i: (0, i, 0)),
            pl.BlockSpec((BR, d), lambda i: (i, 0)),
            pl.BlockSpec((BR, 128), lambda i: (i, 0)),
            pl.BlockSpec((1, d), lambda i: (0, 0)),
            pl.BlockSpec((k, m), lambda i: (0, 0)),
        ],
        out_specs=[
            pl.BlockSpec((BR, d), lambda i: (i, 0)),
            pl.BlockSpec((BR, m), lambda i: (i, 0)),
        ],
        out_shape=[
            jax.ShapeDtypeStruct((NP, d), jnp.float32),
            jax.ShapeDtypeStruct((NP, m), jnp.float32),
        ],
    )(ap, g, dinv, b, wnext)


def _finalize_residual(ap, g, dinv, b, x0):
    def body(ap_ref, g_ref, dinv_ref, b_ref, x0_ref, o_ref):
        ssum = ap_ref[0] + ap_ref[1] + g_ref[...]
        o_ref[...] = jnp.maximum(
            dinv_ref[...] * ssum + b_ref[...] + x0_ref[...], 0.0)

    return pl.pallas_call(
        body,
        grid=(NP // BR,),
        in_specs=[
            pl.BlockSpec((2, BR, 128), lambda i: (0, i, 0)),
            pl.BlockSpec((BR, 128), lambda i: (i, 0)),
            pl.BlockSpec((BR, 128), lambda i: (i, 0)),
            pl.BlockSpec((1, 128), lambda i: (0, 0)),
            pl.BlockSpec((BR, 128), lambda i: (i, 0)),
        ],
        out_specs=pl.BlockSpec((BR, 128), lambda i: (i, 0)),
        out_shape=jax.ShapeDtypeStruct((NP, 128), jnp.float32),
    )(ap, g, dinv, b, x0)


def kernel(x, edge_index, W0, b0, W1, b1, W2, b2):
    src = edge_index[0]
    dst = edge_index[1]
    npad = EP - E
    srcp = jnp.concatenate([src, jnp.full((npad,), N, jnp.int32)])
    dstp = jnp.concatenate([dst, jnp.full((npad,), NP - 1, jnp.int32)])
    xp = jnp.pad(x, ((0, NP - N), (0, 0)))

    zeros_deg = jnp.zeros((NP,), jnp.float32)
    zeros128 = jnp.zeros((RZ, 128), jnp.float32)

    W1p = jnp.pad(W1, ((0, 0), (0, 128 - W1.shape[1])))
    b1p = jnp.pad(b1, (0, 128 - b1.shape[0]))
    W2p = jnp.pad(W2, ((0, 128 - W2.shape[0]), (0, 0)))

    degp = _deg_call(dstp, zeros_deg).T
    h0 = _matmul(xp, W0)
    dinv, g0 = _dinv_scale(degp, h0)

    a0 = _agg_call(g0, srcp, dstp, zeros128).reshape(NC, NP, 128)
    x0, g1 = _finalize_matmul(a0, g0, dinv, b0.reshape(1, 128), W1p)

    a1 = _agg_call(g1, srcp, dstp, zeros128).reshape(NC, NP, 128)
    _, g2 = _finalize_matmul(a1, g1, dinv, b1p.reshape(1, 128), W2p)

    a2 = _agg_call(g2, srcp, dstp, zeros128).reshape(NC, NP, 128)
    out = _finalize_residual(a2, g2, dinv, b2.reshape(1, 128), x0)
    return out[:N]


# speedup vs baseline: 7.2365x; 1.1966x over previous
import dataclasses
import functools

import jax
import jax.numpy as jnp
from jax import lax
from jax.experimental import pallas as pl
from jax.experimental.pallas import tpu as pltpu
from jax.experimental.pallas import tpu_sc as plsc

N = 10000
E = 320000
NP = 10240
NC = 2
NS = 16
NW = NC * NS
RZ = NP // NS
C = 128
CH = 80
EP = NW * CH * C
BR = 512

_mesh = plsc.VectorSubcoreMesh(core_axis_name="c", subcore_axis_name="s")

_sc_params = pltpu.CompilerParams()
if "needs_layout_passes" in pltpu.CompilerParams.__dataclass_fields__:
    _sc_params = dataclasses.replace(_sc_params, needs_layout_passes=False)


def _make_deg():
    @functools.partial(
        pl.kernel,
        out_type=jax.ShapeDtypeStruct((NW, NP), jnp.float32),
        mesh=_mesh,
        compiler_params=_sc_params,
        scratch_types=[
            pltpu.VMEM((C,), jnp.int32),
            pltpu.VMEM((NP,), jnp.float32),
            pltpu.SemaphoreType.DMA,
        ],
    )
    def deg_kernel(dst_hbm, zeros_hbm, out_hbm, dstv, degloc, sem):
        c = lax.axis_index("c")
        s = lax.axis_index("s")
        w = c * NS + s
        pltpu.sync_copy(zeros_hbm, degloc)
        ones16 = jnp.ones((16,), jnp.float32)

        @pl.loop(0, CH)
        def _(j):
            base = (w * CH + j) * C
            pltpu.sync_copy(dst_hbm.at[pl.ds(base, C)], dstv)
            for k in range(C // 16):
                idx = dstv[pl.ds(k * 16, 16)]
                plsc.addupdate_scatter(degloc, [idx], ones16)

        pltpu.sync_copy(degloc, out_hbm.at[w])

    return deg_kernel


def _make_agg(d):
    @functools.partial(
        pl.kernel,
        out_type=jax.ShapeDtypeStruct((NC * NP, d), jnp.float32),
        mesh=_mesh,
        compiler_params=_sc_params,
        scratch_types=[
            pltpu.VMEM((CH * C,), jnp.int32),
            pltpu.VMEM((C,), jnp.int32),
            pltpu.VMEM((C,), jnp.int32),
            pltpu.VMEM((C, d), jnp.float32),
            pltpu.VMEM((C, d), jnp.float32),
            pltpu.VMEM_SHARED((NP, d), jnp.float32),
            pltpu.SemaphoreType.DMA,
            pltpu.SemaphoreType.DMA,
            pltpu.SemaphoreType.DMA,
            pltpu.SemaphoreType.DMA,
            pltpu.SemaphoreType.DMA,
        ],
    )
    def agg_kernel(g_hbm, src_hbm, dst_hbm, zeros_hbm, out_hbm,
                   srcall, dstva, dstvb, bufa, bufb, acc,
                   semga, semgb, sems, semda, semdb):
        c = lax.axis_index("c")
        s = lax.axis_index("s")
        w = c * NS + s
        ebase = w * CH * C
        pltpu.async_copy(src_hbm.at[pl.ds(ebase, CH * C)], srcall, semga)
        pltpu.sync_copy(zeros_hbm, acc.at[pl.ds(s * RZ, RZ)])
        pltpu.make_async_copy(src_hbm.at[pl.ds(ebase, CH * C)], srcall,
                              semga).wait()
        plsc.subcore_barrier()

        def src_slice(j):
            return srcall.at[pl.ds(j * C, C)]

        def dst_slice(j):
            return dst_hbm.at[pl.ds(ebase + j * C, C)]

        pltpu.async_copy(dst_slice(0), dstva, semda)
        pltpu.async_copy(dst_slice(1), dstvb, semdb)
        pltpu.async_copy(g_hbm.at[src_slice(0)], bufa, semga)

        @pl.loop(0, CH // 2)
        def _(i):
            a = 2 * i
            b = a + 1
            pltpu.make_async_copy(dst_slice(a), dstva, semda).wait()
            pltpu.make_async_copy(g_hbm.at[src_slice(a)], bufa, semga).wait()
            pltpu.async_copy(g_hbm.at[src_slice(b)], bufb, semgb)
            hs = pltpu.async_copy(bufa, acc.at[dstva], sems, add=True)
            pltpu.make_async_copy(dst_slice(b), dstvb, semdb).wait()
            pltpu.make_async_copy(g_hbm.at[src_slice(b)], bufb, semgb).wait()
            hs.wait()

            @pl.when(i < CH // 2 - 1)
            def _():
                pltpu.async_copy(g_hbm.at[src_slice(a + 2)], bufa, semga)
                pltpu.async_copy(dst_slice(a + 2), dstva, semda)

            pltpu.sync_copy(bufb, acc.at[dstvb], add=True)

            @pl.when(i < CH // 2 - 1)
            def _():
                pltpu.async_copy(dst_slice(b + 2), dstvb, semdb)

        plsc.subcore_barrier()
        pltpu.sync_copy(acc.at[pl.ds(s * RZ, RZ)],
                        out_hbm.at[pl.ds(c * NP + s * RZ, RZ)])

    return agg_kernel


_deg_call = _make_deg()
_agg_call = _make_agg(128)


def _matmul(x, w):
    k, m = w.shape

    def body(x_ref, w_ref, o_ref):
        o_ref[...] = jnp.dot(x_ref[...], w_ref[...],
                             preferred_element_type=jnp.float32)

    return pl.pallas_call(
        body,
        grid=(NP // BR,),
        in_specs=[
            pl.BlockSpec((BR, k), lambda i: (i, 0)),
            pl.BlockSpec((k, m), lambda i: (0, 0)),
        ],
        out_specs=pl.BlockSpec((BR, m), lambda i: (i, 0)),
        out_shape=jax.ShapeDtypeStruct((NP, m), jnp.float32),
    )(x, w)


def _dinv_scale(degp, h0):
    def body(degp_ref, h_ref, dinv_ref, g_ref):
        deg = jnp.sum(degp_ref[...], axis=1, keepdims=True) + 1.0
        dinv = lax.rsqrt(deg)
        dinv_ref[...] = jnp.broadcast_to(dinv, (BR, 128))
        g_ref[...] = dinv * h_ref[...]

    return pl.pallas_call(
        body,
        grid=(NP // BR,),
        in_specs=[
            pl.BlockSpec((BR, NW), lambda i: (i, 0)),
            pl.BlockSpec((BR, 128), lambda i: (i, 0)),
        ],
        out_specs=[
            pl.BlockSpec((BR, 128), lambda i: (i, 0)),
            pl.BlockSpec((BR, 128), lambda i: (i, 0)),
        ],
        out_shape=[jax.ShapeDtypeStruct((NP, 128), jnp.float32)] * 2,
    )(degp, h0)


def _finalize_matmul(ap, g, dinv, b, wnext):
    d = g.shape[1]
    k, m = wnext.shape

    def body(ap_ref, g_ref, dinv_ref, b_ref, w_ref, x_ref, gn_ref):
        ssum = ap_ref[0] + ap_ref[1] + g_ref[...]
        xl = jnp.maximum(dinv_ref[:, :d] * ssum + b_ref[...], 0.0)
        x_ref[...] = xl
        gn_ref[...] = dinv_ref[:, :m] * jnp.dot(
            xl, w_ref[...], preferred_element_type=jnp.float32)

    return pl.pallas_call(
        body,
        grid=(NP // BR,),
        in_specs=[
            pl.BlockSpec((2, BR, d), lambda i: (0, i, 0)),
            pl.BlockSpec((BR, d), lambda i: (i, 0)),
            pl.BlockSpec((BR, 128), lambda i: (i, 0)),
            pl.BlockSpec((1, d), lambda i: (0, 0)),
            pl.BlockSpec((k, m), lambda i: (0, 0)),
        ],
        out_specs=[
            pl.BlockSpec((BR, d), lambda i: (i, 0)),
            pl.BlockSpec((BR, m), lambda i: (i, 0)),
        ],
        out_shape=[
            jax.ShapeDtypeStruct((NP, d), jnp.float32),
            jax.ShapeDtypeStruct((NP, m), jnp.float32),
        ],
    )(ap, g, dinv, b, wnext)


def _finalize_residual(ap, g, dinv, b, x0):
    def body(ap_ref, g_ref, dinv_ref, b_ref, x0_ref, o_ref):
        ssum = ap_ref[0] + ap_ref[1] + g_ref[...]
        o_ref[...] = jnp.maximum(
            dinv_ref[...] * ssum + b_ref[...] + x0_ref[...], 0.0)

    return pl.pallas_call(
        body,
        grid=(NP // BR,),
        in_specs=[
            pl.BlockSpec((2, BR, 128), lambda i: (0, i, 0)),
            pl.BlockSpec((BR, 128), lambda i: (i, 0)),
            pl.BlockSpec((BR, 128), lambda i: (i, 0)),
            pl.BlockSpec((1, 128), lambda i: (0, 0)),
            pl.BlockSpec((BR, 128), lambda i: (i, 0)),
        ],
        out_specs=pl.BlockSpec((BR, 128), lambda i: (i, 0)),
        out_shape=jax.ShapeDtypeStruct((NP, 128), jnp.float32),
    )(ap, g, dinv, b, x0)


def kernel(x, edge_index, W0, b0, W1, b1, W2, b2):
    src = edge_index[0]
    dst = edge_index[1]
    npad = EP - E
    srcp = jnp.concatenate([src, jnp.full((npad,), N, jnp.int32)])
    dstp = jnp.concatenate([dst, jnp.full((npad,), NP - 1, jnp.int32)])
    xp = jnp.pad(x, ((0, NP - N), (0, 0)))

    zeros_deg = jnp.zeros((NP,), jnp.float32)
    zeros128 = jnp.zeros((RZ, 128), jnp.float32)

    W1p = jnp.pad(W1, ((0, 0), (0, 128 - W1.shape[1])))
    b1p = jnp.pad(b1, (0, 128 - b1.shape[0]))
    W2p = jnp.pad(W2, ((0, 128 - W2.shape[0]), (0, 0)))

    degp = _deg_call(dstp, zeros_deg).T
    h0 = _matmul(xp, W0)
    dinv, g0 = _dinv_scale(degp, h0)

    a0 = _agg_call(g0, srcp, dstp, zeros128).reshape(NC, NP, 128)
    x0, g1 = _finalize_matmul(a0, g0, dinv, b0.reshape(1, 128), W1p)

    a1 = _agg_call(g1, srcp, dstp, zeros128).reshape(NC, NP, 128)
    _, g2 = _finalize_matmul(a1, g1, dinv, b1p.reshape(1, 128), W2p)

    a2 = _agg_call(g2, srcp, dstp, zeros128).reshape(NC, NP, 128)
    out = _finalize_residual(a2, g2, dinv, b2.reshape(1, 128), x0)
    return out[:N]


# speedup vs baseline: 21.8313x; 3.0169x over previous
import dataclasses
import functools

import jax
import jax.numpy as jnp
from jax import lax
from jax.experimental import pallas as pl
from jax.experimental.pallas import tpu as pltpu
from jax.experimental.pallas import tpu_sc as plsc

N = 10000
E = 320000
NP = 10240
NC = 2
NS = 16
NW = NC * NS
RZ = NP // NS
C = 128
CH = 80
EP = NW * CH * C
BR = 512

_mesh = plsc.VectorSubcoreMesh(core_axis_name="c", subcore_axis_name="s")

_sc_params = pltpu.CompilerParams()
if "needs_layout_passes" in pltpu.CompilerParams.__dataclass_fields__:
    _sc_params = dataclasses.replace(_sc_params, needs_layout_passes=False)


def _make_deg():
    @functools.partial(
        pl.kernel,
        out_type=jax.ShapeDtypeStruct((NW, NP), jnp.float32),
        mesh=_mesh,
        compiler_params=_sc_params,
        scratch_types=[
            pltpu.VMEM((C,), jnp.int32),
            pltpu.VMEM((NP,), jnp.float32),
            pltpu.SemaphoreType.DMA,
        ],
    )
    def deg_kernel(dst_hbm, zeros_hbm, out_hbm, dstv, degloc, sem):
        c = lax.axis_index("c")
        s = lax.axis_index("s")
        w = c * NS + s
        pltpu.sync_copy(zeros_hbm, degloc)
        ones16 = jnp.ones((16,), jnp.float32)

        @pl.loop(0, CH)
        def _(j):
            base = (w * CH + j) * C
            pltpu.sync_copy(dst_hbm.at[pl.ds(base, C)], dstv)
            for k in range(C // 16):
                idx = dstv[pl.ds(k * 16, 16)]
                plsc.addupdate_scatter(degloc, [idx], ones16)

        pltpu.sync_copy(degloc, out_hbm.at[w])

    return deg_kernel


def _make_agg(d):
    @functools.partial(
        pl.kernel,
        out_type=jax.ShapeDtypeStruct((NC * NP, d), jnp.float32),
        mesh=_mesh,
        compiler_params=_sc_params,
        scratch_types=[
            pltpu.VMEM((CH * C,), jnp.int32),
            pltpu.VMEM((C,), jnp.int32),
            pltpu.VMEM((C,), jnp.int32),
            pltpu.VMEM((C, d), jnp.float32),
            pltpu.VMEM((C, d), jnp.float32),
            pltpu.VMEM_SHARED((NP, d), jnp.float32),
            pltpu.SemaphoreType.DMA,
            pltpu.SemaphoreType.DMA,
            pltpu.SemaphoreType.DMA,
            pltpu.SemaphoreType.DMA,
            pltpu.SemaphoreType.DMA,
        ],
    )
    def agg_kernel(g_hbm, src_hbm, dst_hbm, zeros_hbm, out_hbm,
                   srcall, dstva, dstvb, bufa, bufb, acc,
                   semga, semgb, sems, semda, semdb):
        c = lax.axis_index("c")
        s = lax.axis_index("s")
        w = c * NS + s
        ebase = w * CH * C
        pltpu.async_copy(src_hbm.at[pl.ds(ebase, CH * C)], srcall, semga)
        pltpu.sync_copy(zeros_hbm, acc.at[pl.ds(s * RZ, RZ)])
        pltpu.make_async_copy(src_hbm.at[pl.ds(ebase, CH * C)], srcall,
                              semga).wait()
        plsc.subcore_barrier()

        def src_slice(j):
            return srcall.at[pl.ds(j * C, C)]

        def dst_slice(j):
            return dst_hbm.at[pl.ds(ebase + j * C, C)]

        pltpu.async_copy(dst_slice(0), dstva, semda)
        pltpu.async_copy(dst_slice(1), dstvb, semdb)
        pltpu.async_copy(g_hbm.at[src_slice(0)], bufa, semga)

        @pl.loop(0, CH // 2)
        def _(i):
            a = 2 * i
            b = a + 1
            pltpu.make_async_copy(dst_slice(a), dstva, semda).wait()
            pltpu.make_async_copy(g_hbm.at[src_slice(a)], bufa, semga).wait()
            pltpu.async_copy(g_hbm.at[src_slice(b)], bufb, semgb)
            hs = pltpu.async_copy(bufa, acc.at[dstva], sems, add=True)
            pltpu.make_async_copy(dst_slice(b), dstvb, semdb).wait()
            pltpu.make_async_copy(g_hbm.at[src_slice(b)], bufb, semgb).wait()
            hs.wait()

            @pl.when(i < CH // 2 - 1)
            def _():
                pltpu.async_copy(g_hbm.at[src_slice(a + 2)], bufa, semga)
                pltpu.async_copy(dst_slice(a + 2), dstva, semda)

            pltpu.sync_copy(bufb, acc.at[dstvb], add=True)

            @pl.when(i < CH // 2 - 1)
            def _():
                pltpu.async_copy(dst_slice(b + 2), dstvb, semdb)

        plsc.subcore_barrier()
        pltpu.sync_copy(acc.at[pl.ds(s * RZ, RZ)],
                        out_hbm.at[pl.ds(c * NP + s * RZ, RZ)])

    return agg_kernel


_deg_call = _make_deg()
_agg_call = _make_agg(128)


def _matmul(x, w):
    k, m = w.shape

    def body(x_ref, w_ref, o_ref):
        o_ref[...] = jnp.dot(x_ref[...], w_ref[...],
                             preferred_element_type=jnp.float32)

    return pl.pallas_call(
        body,
        grid=(NP // BR,),
        in_specs=[
            pl.BlockSpec((BR, k), lambda i: (i, 0)),
            pl.BlockSpec((k, m), lambda i: (0, 0)),
        ],
        out_specs=pl.BlockSpec((BR, m), lambda i: (i, 0)),
        out_shape=jax.ShapeDtypeStruct((NP, m), jnp.float32),
    )(x, w)


def _dinv_scale(degp, h0):
    def body(degp_ref, h_ref, dinv_ref, g_ref):
        deg = jnp.sum(degp_ref[...], axis=1, keepdims=True) + 1.0
        dinv = lax.rsqrt(deg)
        dinv_ref[...] = jnp.broadcast_to(dinv, (BR, 128))
        g_ref[...] = dinv * h_ref[...]

    return pl.pallas_call(
        body,
        grid=(NP // BR,),
        in_specs=[
            pl.BlockSpec((BR, NW), lambda i: (i, 0)),
            pl.BlockSpec((BR, 128), lambda i: (i, 0)),
        ],
        out_specs=[
            pl.BlockSpec((BR, 128), lambda i: (i, 0)),
            pl.BlockSpec((BR, 128), lambda i: (i, 0)),
        ],
        out_shape=[jax.ShapeDtypeStruct((NP, 128), jnp.float32)] * 2,
    )(degp, h0)


def _finalize_matmul(ap, g, dinv, b, wnext):
    d = g.shape[1]
    k, m = wnext.shape

    def body(ap_ref, g_ref, dinv_ref, b_ref, w_ref, x_ref, gn_ref):
        ssum = ap_ref[0] + ap_ref[1] + g_ref[...]
        xl = jnp.maximum(dinv_ref[:, :d] * ssum + b_ref[...], 0.0)
        x_ref[...] = xl
        gn_ref[...] = dinv_ref[:, :m] * jnp.dot(
            xl, w_ref[...], preferred_element_type=jnp.float32)

    return pl.pallas_call(
        body,
        grid=(NP // BR,),
        in_specs=[
            pl.BlockSpec((2, BR, d), lambda i: (0, i, 0)),
            pl.BlockSpec((BR, d), lambda i: (i, 0)),
            pl.BlockSpec((BR, 128), lambda i: (i, 0)),
            pl.BlockSpec((1, d), lambda i: (0, 0)),
            pl.BlockSpec((k, m), lambda i: (0, 0)),
        ],
        out_specs=[
            pl.BlockSpec((BR, d), lambda i: (i, 0)),
            pl.BlockSpec((BR, m), lambda i: (i, 0)),
        ],
        out_shape=[
            jax.ShapeDtypeStruct((NP, d), jnp.float32),
            jax.ShapeDtypeStruct((NP, m), jnp.float32),
        ],
    )(ap, g, dinv, b, wnext)


def _finalize_residual(ap, g, dinv, b, x0):
    def body(ap_ref, g_ref, dinv_ref, b_ref, x0_ref, o_ref):
        ssum = ap_ref[0] + ap_ref[1] + g_ref[...]
        o_ref[...] = jnp.maximum(
            dinv_ref[...] * ssum + b_ref[...] + x0_ref[...], 0.0)

    return pl.pallas_call(
        body,
        grid=(NP // BR,),
        in_specs=[
            pl.BlockSpec((2, BR, 128), lambda i: (0, i, 0)),
            pl.BlockSpec((BR, 128), lambda i: (i, 0)),
            pl.BlockSpec((BR, 128), lambda i: (i, 0)),
            pl.BlockSpec((1, 128), lambda i: (0, 0)),
            pl.BlockSpec((BR, 128), lambda i: (i, 0)),
        ],
        out_specs=pl.BlockSpec((BR, 128), lambda i: (i, 0)),
        out_shape=jax.ShapeDtypeStruct((NP, 128), jnp.float32),
    )(ap, g, dinv, b, x0)


def kernel(x, edge_index, W0, b0, W1, b1, W2, b2):
    src = edge_index[0]
    dst = edge_index[1]
    npad = EP - E
    spread = N + (jnp.arange(npad, dtype=jnp.int32) % (NP - N))
    srcp = jnp.concatenate([src, spread])
    dstp = jnp.concatenate([dst, spread])
    xp = jnp.pad(x, ((0, NP - N), (0, 0)))

    zeros_deg = jnp.zeros((NP,), jnp.float32)
    zeros128 = jnp.zeros((RZ, 128), jnp.float32)

    W1p = jnp.pad(W1, ((0, 0), (0, 128 - W1.shape[1])))
    b1p = jnp.pad(b1, (0, 128 - b1.shape[0]))
    W2p = jnp.pad(W2, ((0, 128 - W2.shape[0]), (0, 0)))

    degp = _deg_call(dstp, zeros_deg).T
    h0 = _matmul(xp, W0)
    dinv, g0 = _dinv_scale(degp, h0)

    a0 = _agg_call(g0, srcp, dstp, zeros128).reshape(NC, NP, 128)
    x0, g1 = _finalize_matmul(a0, g0, dinv, b0.reshape(1, 128), W1p)

    a1 = _agg_call(g1, srcp, dstp, zeros128).reshape(NC, NP, 128)
    _, g2 = _finalize_matmul(a1, g1, dinv, b1p.reshape(1, 128), W2p)

    a2 = _agg_call(g2, srcp, dstp, zeros128).reshape(NC, NP, 128)
    out = _finalize_residual(a2, g2, dinv, b2.reshape(1, 128), x0)
    return out[:N]


# speedup vs baseline: 23.5038x; 1.0766x over previous
import dataclasses
import functools

import jax
import jax.numpy as jnp
from jax import lax
from jax.experimental import pallas as pl
from jax.experimental.pallas import tpu as pltpu
from jax.experimental.pallas import tpu_sc as plsc

N = 10000
E = 320000
NP = 10240
NC = 2
NS = 16
NW = NC * NS
RZ = NP // NS
C = 128
CH = 80
EP = NW * CH * C
BR = 512

_mesh = plsc.VectorSubcoreMesh(core_axis_name="c", subcore_axis_name="s")

_sc_params = pltpu.CompilerParams()
if "needs_layout_passes" in pltpu.CompilerParams.__dataclass_fields__:
    _sc_params = dataclasses.replace(_sc_params, needs_layout_passes=False)


def _make_deg():
    @functools.partial(
        pl.kernel,
        out_type=jax.ShapeDtypeStruct((NW, NP), jnp.float32),
        mesh=_mesh,
        compiler_params=_sc_params,
        scratch_types=[
            pltpu.VMEM((CH * C,), jnp.int32),
            pltpu.VMEM((NP,), jnp.float32),
            pltpu.SemaphoreType.DMA,
        ],
    )
    def deg_kernel(dst_hbm, zeros_hbm, out_hbm, dstall, degloc, sem):
        c = lax.axis_index("c")
        s = lax.axis_index("s")
        w = c * NS + s
        ebase = w * CH * C
        pltpu.async_copy(dst_hbm.at[pl.ds(ebase, CH * C)], dstall, sem)
        pltpu.sync_copy(zeros_hbm, degloc)
        pltpu.make_async_copy(dst_hbm.at[pl.ds(ebase, CH * C)], dstall,
                              sem).wait()
        ones16 = jnp.ones((16,), jnp.float32)

        @pl.loop(0, CH)
        def _(j):
            for k in range(C // 16):
                idx = dstall[pl.ds(j * C + k * 16, 16)]
                plsc.addupdate_scatter(degloc, [idx], ones16)

        pltpu.sync_copy(degloc, out_hbm.at[w])

    return deg_kernel


def _make_agg(d):
    @functools.partial(
        pl.kernel,
        out_type=jax.ShapeDtypeStruct((NC * NP, d), jnp.float32),
        mesh=_mesh,
        compiler_params=_sc_params,
        scratch_types=[
            pltpu.VMEM((CH * C,), jnp.int32),
            pltpu.VMEM((C,), jnp.int32),
            pltpu.VMEM((C,), jnp.int32),
            pltpu.VMEM((C, d), jnp.float32),
            pltpu.VMEM((C, d), jnp.float32),
            pltpu.VMEM_SHARED((NP, d), jnp.float32),
            pltpu.SemaphoreType.DMA,
            pltpu.SemaphoreType.DMA,
            pltpu.SemaphoreType.DMA,
            pltpu.SemaphoreType.DMA,
            pltpu.SemaphoreType.DMA,
        ],
    )
    def agg_kernel(g_hbm, src_hbm, dst_hbm, zeros_hbm, out_hbm,
                   srcall, dstva, dstvb, bufa, bufb, acc,
                   semga, semgb, sems, semda, semdb):
        c = lax.axis_index("c")
        s = lax.axis_index("s")
        w = c * NS + s
        ebase = w * CH * C
        pltpu.async_copy(src_hbm.at[pl.ds(ebase, CH * C)], srcall, semga)
        pltpu.sync_copy(zeros_hbm, acc.at[pl.ds(s * RZ, RZ)])
        pltpu.make_async_copy(src_hbm.at[pl.ds(ebase, CH * C)], srcall,
                              semga).wait()
        plsc.subcore_barrier()

        def src_slice(j):
            return srcall.at[pl.ds(j * C, C)]

        def dst_slice(j):
            return dst_hbm.at[pl.ds(ebase + j * C, C)]

        pltpu.async_copy(dst_slice(0), dstva, semda)
        pltpu.async_copy(dst_slice(1), dstvb, semdb)
        pltpu.async_copy(g_hbm.at[src_slice(0)], bufa, semga)

        @pl.loop(0, CH // 2)
        def _(i):
            a = 2 * i
            b = a + 1
            pltpu.make_async_copy(dst_slice(a), dstva, semda).wait()
            pltpu.make_async_copy(g_hbm.at[src_slice(a)], bufa, semga).wait()
            pltpu.async_copy(g_hbm.at[src_slice(b)], bufb, semgb)
            hs = pltpu.async_copy(bufa, acc.at[dstva], sems, add=True)
            pltpu.make_async_copy(dst_slice(b), dstvb, semdb).wait()
            pltpu.make_async_copy(g_hbm.at[src_slice(b)], bufb, semgb).wait()
            hs.wait()

            @pl.when(i < CH // 2 - 1)
            def _():
                pltpu.async_copy(g_hbm.at[src_slice(a + 2)], bufa, semga)
                pltpu.async_copy(dst_slice(a + 2), dstva, semda)

            pltpu.sync_copy(bufb, acc.at[dstvb], add=True)

            @pl.when(i < CH // 2 - 1)
            def _():
                pltpu.async_copy(dst_slice(b + 2), dstvb, semdb)

        plsc.subcore_barrier()
        pltpu.sync_copy(acc.at[pl.ds(s * RZ, RZ)],
                        out_hbm.at[pl.ds(c * NP + s * RZ, RZ)])

    return agg_kernel


_deg_call = _make_deg()
_agg_call = _make_agg(128)


def _matmul_scale(degp, x, w):
    def body(degp_ref, x_ref, w_ref, dinv_ref, g_ref):
        deg = jnp.sum(degp_ref[...], axis=1, keepdims=True) + 1.0
        dinv = lax.rsqrt(deg)
        dinv_ref[...] = jnp.broadcast_to(dinv, (BR, 128))
        g_ref[...] = dinv * jnp.dot(x_ref[...], w_ref[...],
                                    preferred_element_type=jnp.float32)

    return pl.pallas_call(
        body,
        grid=(NP // BR,),
        in_specs=[
            pl.BlockSpec((BR, NW), lambda i: (i, 0)),
            pl.BlockSpec((BR, 128), lambda i: (i, 0)),
            pl.BlockSpec((128, 128), lambda i: (0, 0)),
        ],
        out_specs=[
            pl.BlockSpec((BR, 128), lambda i: (i, 0)),
            pl.BlockSpec((BR, 128), lambda i: (i, 0)),
        ],
        out_shape=[jax.ShapeDtypeStruct((NP, 128), jnp.float32)] * 2,
    )(degp, x, w)


def _finalize_matmul(ap, g, dinv, b, wnext):
    d = g.shape[1]
    k, m = wnext.shape

    def body(ap_ref, g_ref, dinv_ref, b_ref, w_ref, x_ref, gn_ref):
        ssum = ap_ref[0] + ap_ref[1] + g_ref[...]
        xl = jnp.maximum(dinv_ref[:, :d] * ssum + b_ref[...], 0.0)
        x_ref[...] = xl
        gn_ref[...] = dinv_ref[:, :m] * jnp.dot(
            xl, w_ref[...], preferred_element_type=jnp.float32)

    return pl.pallas_call(
        body,
        grid=(NP // BR,),
        in_specs=[
            pl.BlockSpec((2, BR, d), lambda i: (0, i, 0)),
            pl.BlockSpec((BR, d), lambda i: (i, 0)),
            pl.BlockSpec((BR, 128), lambda i: (i, 0)),
            pl.BlockSpec((1, d), lambda i: (0, 0)),
            pl.BlockSpec((k, m), lambda i: (0, 0)),
        ],
        out_specs=[
            pl.BlockSpec((BR, d), lambda i: (i, 0)),
            pl.BlockSpec((BR, m), lambda i: (i, 0)),
        ],
        out_shape=[
            jax.ShapeDtypeStruct((NP, d), jnp.float32),
            jax.ShapeDtypeStruct((NP, m), jnp.float32),
        ],
    )(ap, g, dinv, b, wnext)


def _finalize_residual(ap, g, dinv, b, x0):
    BN = 400

    def body(ap_ref, g_ref, dinv_ref, b_ref, x0_ref, o_ref):
        ssum = ap_ref[0] + ap_ref[1] + g_ref[...]
        o_ref[...] = jnp.maximum(
            dinv_ref[...] * ssum + b_ref[...] + x0_ref[...], 0.0)

    return pl.pallas_call(
        body,
        grid=(N // BN,),
        in_specs=[
            pl.BlockSpec((2, BN, 128), lambda i: (0, i, 0)),
            pl.BlockSpec((BN, 128), lambda i: (i, 0)),
            pl.BlockSpec((BN, 128), lambda i: (i, 0)),
            pl.BlockSpec((1, 128), lambda i: (0, 0)),
            pl.BlockSpec((BN, 128), lambda i: (i, 0)),
        ],
        out_specs=pl.BlockSpec((BN, 128), lambda i: (i, 0)),
        out_shape=jax.ShapeDtypeStruct((N, 128), jnp.float32),
    )(ap, g, dinv, b, x0)


def kernel(x, edge_index, W0, b0, W1, b1, W2, b2):
    src = edge_index[0]
    dst = edge_index[1]
    npad = EP - E
    spread = N + (jnp.arange(npad, dtype=jnp.int32) % (NP - N))
    srcp = jnp.concatenate([src, spread])
    dstp = jnp.concatenate([dst, spread])
    xp = jnp.pad(x, ((0, NP - N), (0, 0)))

    zeros_deg = jnp.zeros((NP,), jnp.float32)
    zeros128 = jnp.zeros((RZ, 128), jnp.float32)

    W1p = jnp.pad(W1, ((0, 0), (0, 128 - W1.shape[1])))
    b1p = jnp.pad(b1, (0, 128 - b1.shape[0]))
    W2p = jnp.pad(W2, ((0, 128 - W2.shape[0]), (0, 0)))

    degp = _deg_call(dstp, zeros_deg).T
    dinv, g0 = _matmul_scale(degp, xp, W0)

    a0 = _agg_call(g0, srcp, dstp, zeros128).reshape(NC, NP, 128)
    x0, g1 = _finalize_matmul(a0, g0, dinv, b0.reshape(1, 128), W1p)

    a1 = _agg_call(g1, srcp, dstp, zeros128).reshape(NC, NP, 128)
    _, g2 = _finalize_matmul(a1, g1, dinv, b1p.reshape(1, 128), W2p)

    a2 = _agg_call(g2, srcp, dstp, zeros128).reshape(NC, NP, 128)
    return _finalize_residual(a2, g2, dinv, b2.reshape(1, 128), x0)


# speedup vs baseline: 23.8023x; 1.0127x over previous
import dataclasses
import functools

import jax
import jax.numpy as jnp
from jax import lax
from jax.experimental import pallas as pl
from jax.experimental.pallas import tpu as pltpu
from jax.experimental.pallas import tpu_sc as plsc

N = 10000
E = 320000
NP = 10240
NC = 2
NS = 16
NW = NC * NS
RZ = NP // NS
C = 128
CH = 80
EP = NW * CH * C
BR = 512

_mesh = plsc.VectorSubcoreMesh(core_axis_name="c", subcore_axis_name="s")

_sc_params = pltpu.CompilerParams()
if "needs_layout_passes" in pltpu.CompilerParams.__dataclass_fields__:
    _sc_params = dataclasses.replace(_sc_params, needs_layout_passes=False)


def _make_deg():
    @functools.partial(
        pl.kernel,
        out_type=jax.ShapeDtypeStruct((NW, NP), jnp.float32),
        mesh=_mesh,
        compiler_params=_sc_params,
        scratch_types=[
            pltpu.VMEM((CH * C,), jnp.int32),
            pltpu.VMEM((NP,), jnp.float32),
            pltpu.SemaphoreType.DMA,
        ],
    )
    def deg_kernel(dst_hbm, zeros_hbm, out_hbm, dstall, degloc, sem):
        c = lax.axis_index("c")
        s = lax.axis_index("s")
        w = c * NS + s
        ebase = w * CH * C
        pltpu.async_copy(dst_hbm.at[pl.ds(ebase, CH * C)], dstall, sem)
        pltpu.sync_copy(zeros_hbm, degloc)
        pltpu.make_async_copy(dst_hbm.at[pl.ds(ebase, CH * C)], dstall,
                              sem).wait()
        ones16 = jnp.ones((16,), jnp.float32)

        @pl.loop(0, CH)
        def _(j):
            for k in range(C // 16):
                idx = dstall[pl.ds(j * C + k * 16, 16)]
                plsc.addupdate_scatter(degloc, [idx], ones16)

        pltpu.sync_copy(degloc, out_hbm.at[w])

    return deg_kernel


def _make_agg(d):
    @functools.partial(
        pl.kernel,
        out_type=jax.ShapeDtypeStruct((NC * NP, d), jnp.float32),
        mesh=_mesh,
        compiler_params=_sc_params,
        scratch_types=[
            pltpu.VMEM((CH * C,), jnp.int32),
            pltpu.VMEM((C,), jnp.int32),
            pltpu.VMEM((C,), jnp.int32),
            pltpu.VMEM((C, d), jnp.float32),
            pltpu.VMEM((C, d), jnp.float32),
            pltpu.VMEM_SHARED((NP, d), jnp.float32),
            pltpu.SemaphoreType.DMA,
            pltpu.SemaphoreType.DMA,
            pltpu.SemaphoreType.DMA,
            pltpu.SemaphoreType.DMA,
            pltpu.SemaphoreType.DMA,
        ],
    )
    def agg_kernel(g_hbm, src_hbm, dst_hbm, zeros_hbm, out_hbm,
                   srcall, dstva, dstvb, bufa, bufb, acc,
                   semga, semgb, sems, semda, semdb):
        c = lax.axis_index("c")
        s = lax.axis_index("s")
        w = c * NS + s
        ebase = w * CH * C
        pltpu.async_copy(src_hbm.at[pl.ds(ebase, CH * C)], srcall, semga)
        pltpu.sync_copy(zeros_hbm, acc.at[pl.ds(s * RZ, RZ)])
        pltpu.make_async_copy(src_hbm.at[pl.ds(ebase, CH * C)], srcall,
                              semga).wait()
        plsc.subcore_barrier()

        def src_slice(j):
            return srcall.at[pl.ds(j * C, C)]

        def dst_slice(j):
            return dst_hbm.at[pl.ds(ebase + j * C, C)]

        pltpu.async_copy(dst_slice(0), dstva, semda)
        pltpu.async_copy(dst_slice(1), dstvb, semdb)
        pltpu.async_copy(g_hbm.at[src_slice(0)], bufa, semga)

        @pl.loop(0, CH // 2)
        def _(i):
            a = 2 * i
            b = a + 1
            pltpu.make_async_copy(dst_slice(a), dstva, semda).wait()
            pltpu.make_async_copy(g_hbm.at[src_slice(a)], bufa, semga).wait()
            pltpu.async_copy(g_hbm.at[src_slice(b)], bufb, semgb)
            hs = pltpu.async_copy(bufa, acc.at[dstva], sems, add=True)
            pltpu.make_async_copy(dst_slice(b), dstvb, semdb).wait()
            pltpu.make_async_copy(g_hbm.at[src_slice(b)], bufb, semgb).wait()
            hs.wait()

            @pl.when(i < CH // 2 - 1)
            def _():
                pltpu.async_copy(g_hbm.at[src_slice(a + 2)], bufa, semga)
                pltpu.async_copy(dst_slice(a + 2), dstva, semda)

            pltpu.sync_copy(bufb, acc.at[dstvb], add=True)

            @pl.when(i < CH // 2 - 1)
            def _():
                pltpu.async_copy(dst_slice(b + 2), dstvb, semdb)

        plsc.subcore_barrier()
        pltpu.sync_copy(acc.at[pl.ds(s * RZ, RZ)],
                        out_hbm.at[pl.ds(c * NP + s * RZ, RZ)])

    return agg_kernel


_deg_call = _make_deg()
_agg_call = _make_agg(128)


def _matmul_scale(degp, x, w):
    def body(degp_ref, x_ref, w_ref, dinv_ref, g_ref):
        deg = jnp.sum(degp_ref[...], axis=0)[:, None] + 1.0
        dinv = lax.rsqrt(deg)
        dinv_ref[...] = jnp.broadcast_to(dinv, (BR, 128))
        g_ref[...] = dinv * jnp.dot(x_ref[...], w_ref[...],
                                    preferred_element_type=jnp.float32)

    return pl.pallas_call(
        body,
        grid=(NP // BR,),
        in_specs=[
            pl.BlockSpec((NW, BR), lambda i: (0, i)),
            pl.BlockSpec((BR, 128), lambda i: (i, 0)),
            pl.BlockSpec((128, 128), lambda i: (0, 0)),
        ],
        out_specs=[
            pl.BlockSpec((BR, 128), lambda i: (i, 0)),
            pl.BlockSpec((BR, 128), lambda i: (i, 0)),
        ],
        out_shape=[jax.ShapeDtypeStruct((NP, 128), jnp.float32)] * 2,
    )(degp, x, w)


def _finalize_matmul(ap, g, dinv, b, wnext):
    d = g.shape[1]
    k, m = wnext.shape

    def body(ap_ref, g_ref, dinv_ref, b_ref, w_ref, x_ref, gn_ref):
        ssum = ap_ref[0] + ap_ref[1] + g_ref[...]
        xl = jnp.maximum(dinv_ref[:, :d] * ssum + b_ref[...], 0.0)
        x_ref[...] = xl
        gn_ref[...] = dinv_ref[:, :m] * jnp.dot(
            xl, w_ref[...], preferred_element_type=jnp.float32)

    return pl.pallas_call(
        body,
        grid=(NP // BR,),
        in_specs=[
            pl.BlockSpec((2, BR, d), lambda i: (0, i, 0)),
            pl.BlockSpec((BR, d), lambda i: (i, 0)),
            pl.BlockSpec((BR, 128), lambda i: (i, 0)),
            pl.BlockSpec((1, d), lambda i: (0, 0)),
            pl.BlockSpec((k, m), lambda i: (0, 0)),
        ],
        out_specs=[
            pl.BlockSpec((BR, d), lambda i: (i, 0)),
            pl.BlockSpec((BR, m), lambda i: (i, 0)),
        ],
        out_shape=[
            jax.ShapeDtypeStruct((NP, d), jnp.float32),
            jax.ShapeDtypeStruct((NP, m), jnp.float32),
        ],
    )(ap, g, dinv, b, wnext)


def _finalize_residual(ap, g, dinv, b, x0):
    BN = 400

    def body(ap_ref, g_ref, dinv_ref, b_ref, x0_ref, o_ref):
        ssum = ap_ref[0] + ap_ref[1] + g_ref[...]
        o_ref[...] = jnp.maximum(
            dinv_ref[...] * ssum + b_ref[...] + x0_ref[...], 0.0)

    return pl.pallas_call(
        body,
        grid=(N // BN,),
        in_specs=[
            pl.BlockSpec((2, BN, 128), lambda i: (0, i, 0)),
            pl.BlockSpec((BN, 128), lambda i: (i, 0)),
            pl.BlockSpec((BN, 128), lambda i: (i, 0)),
            pl.BlockSpec((1, 128), lambda i: (0, 0)),
            pl.BlockSpec((BN, 128), lambda i: (i, 0)),
        ],
        out_specs=pl.BlockSpec((BN, 128), lambda i: (i, 0)),
        out_shape=jax.ShapeDtypeStruct((N, 128), jnp.float32),
    )(ap, g, dinv, b, x0)


def kernel(x, edge_index, W0, b0, W1, b1, W2, b2):
    src = edge_index[0]
    dst = edge_index[1]
    npad = EP - E
    spread = N + (jnp.arange(npad, dtype=jnp.int32) % (NP - N))
    srcp = jnp.concatenate([src, spread])
    dstp = jnp.concatenate([dst, spread])
    xp = jnp.pad(x, ((0, NP - N), (0, 0)))

    zeros_deg = jnp.zeros((NP,), jnp.float32)
    zeros128 = jnp.zeros((RZ, 128), jnp.float32)

    W1p = jnp.pad(W1, ((0, 0), (0, 128 - W1.shape[1])))
    b1p = jnp.pad(b1, (0, 128 - b1.shape[0]))
    W2p = jnp.pad(W2, ((0, 128 - W2.shape[0]), (0, 0)))

    degp = _deg_call(dstp, zeros_deg)
    dinv, g0 = _matmul_scale(degp, xp, W0)

    a0 = _agg_call(g0, srcp, dstp, zeros128).reshape(NC, NP, 128)
    x0, g1 = _finalize_matmul(a0, g0, dinv, b0.reshape(1, 128), W1p)

    a1 = _agg_call(g1, srcp, dstp, zeros128).reshape(NC, NP, 128)
    _, g2 = _finalize_matmul(a1, g1, dinv, b1p.reshape(1, 128), W2p)

    a2 = _agg_call(g2, srcp, dstp, zeros128).reshape(NC, NP, 128)
    return _finalize_residual(a2, g2, dinv, b2.reshape(1, 128), x0)
